# initial kernel scaffold (unmeasured)
import jax
import jax.numpy as jnp
from jax import lax
from jax.experimental import pallas as pl
from jax.experimental.pallas import tpu as pltpu

N_DEV = 8
N_TOK = 2048
D_MODEL = 1024
E_LOCAL = 8
E_TOTAL = 64
CHUNK = N_TOK // N_DEV


def kernel(x, router_W, route_idx, expert_W):
    def body(x_ref, rw_ref, idx_ref, ew_ref, out_ref,
             comm_ref, stage_ref, send_sems, recv_sems):
        p = lax.axis_index("i")
        left = lax.rem(p + N_DEV - 1, N_DEV)
        right = lax.rem(p + 1, N_DEV)

        barrier = pltpu.get_barrier_semaphore()
        pl.semaphore_signal(barrier, inc=1, device_id=(left,),
                            device_id_type=pl.DeviceIdType.MESH)
        pl.semaphore_signal(barrier, inc=1, device_id=(right,),
                            device_id_type=pl.DeviceIdType.MESH)
        pl.semaphore_wait(barrier, 2)

        x_all = x_ref[:, :]
        scores = jnp.dot(x_all, rw_ref[:, :], preferred_element_type=jnp.float32)
        s_max = jnp.max(scores, axis=-1, keepdims=True)
        probs = jnp.exp(scores - s_max)
        e_ids = lax.broadcasted_iota(jnp.int32, (N_TOK, E_TOTAL), 1)
        mask = jnp.logical_or(e_ids == idx_ref[:, 0:1],
                              e_ids == idx_ref[:, 1:2]).astype(jnp.float32)
        w = probs * mask
        gates = w / jnp.sum(w, axis=-1, keepdims=True)

        row = lax.broadcasted_iota(jnp.int32, (E_TOTAL, E_LOCAL), 0)
        col = lax.broadcasted_iota(jnp.int32, (E_TOTAL, E_LOCAL), 1)
        sel = (row == p * E_LOCAL + col).astype(jnp.float32)
        gates_local = jnp.dot(gates, sel, preferred_element_type=jnp.float32)

        def partial_chunk(c):
            start = c * CHUNK
            xs = x_ref[pl.ds(start, CHUNK), :]
            g = lax.dynamic_slice(gates_local, (start, 0), (CHUNK, E_LOCAL))
            acc = jnp.zeros((CHUNK, D_MODEL), jnp.float32)
            for j in range(E_LOCAL):
                acc = acc + jnp.dot(xs * g[:, j:j + 1], ew_ref[j],
                                    preferred_element_type=jnp.float32)
            return acc

        for s in range(N_DEV - 1):
            c = lax.rem(p + N_DEV - 1 - s, N_DEV)
            val = partial_chunk(c)
            if s == 0:
                stage_ref[:, :] = val
            else:
                stage_ref[:, :] = val + comm_ref[s - 1]
            rdma = pltpu.make_async_remote_copy(
                src_ref=stage_ref,
                dst_ref=comm_ref.at[s],
                send_sem=send_sems.at[s],
                recv_sem=recv_sems.at[s],
                device_id=(right,),
                device_id_type=pl.DeviceIdType.MESH,
            )
            rdma.start()
            rdma.wait()

        out_ref[:, :] = comm_ref[N_DEV - 2] + partial_chunk(p)

    return pl.pallas_call(
        body,
        out_shape=jax.ShapeDtypeStruct((CHUNK, D_MODEL), jnp.float32),
        in_specs=[pl.BlockSpec(memory_space=pltpu.VMEM)] * 4,
        out_specs=pl.BlockSpec(memory_space=pltpu.VMEM),
        scratch_shapes=[
            pltpu.VMEM((N_DEV - 1, CHUNK, D_MODEL), jnp.float32),
            pltpu.VMEM((CHUNK, D_MODEL), jnp.float32),
            pltpu.SemaphoreType.DMA((N_DEV - 1,)),
            pltpu.SemaphoreType.DMA((N_DEV - 1,)),
        ],
        compiler_params=pltpu.CompilerParams(collective_id=0),
    )(x, router_W, route_idx, expert_W)


# baseline (device time: 191191 ns/iter reference)
import jax
import jax.numpy as jnp
from jax import lax
from jax.experimental import pallas as pl
from jax.experimental.pallas import tpu as pltpu

N_DEV = 8
N_TOK = 2048
D_MODEL = 1024
E_LOCAL = 8
E_TOTAL = 64
CHUNK = N_TOK // N_DEV


def kernel(x, router_W, route_idx, expert_W):
    def body(x_ref, rw_ref, idx_ref, ew_ref, out_ref,
             comm_ref, stage_ref, gates_ref, send_sems, recv_sems):
        p = lax.axis_index("i")
        left = lax.rem(p + N_DEV - 1, N_DEV)
        right = lax.rem(p + 1, N_DEV)

        barrier = pltpu.get_barrier_semaphore()
        pl.semaphore_signal(barrier, inc=1, device_id=(left,),
                            device_id_type=pl.DeviceIdType.MESH)
        pl.semaphore_signal(barrier, inc=1, device_id=(right,),
                            device_id_type=pl.DeviceIdType.MESH)
        pl.semaphore_wait(barrier, 2)

        x_all = x_ref[:, :]
        scores = jnp.dot(x_all, rw_ref[:, :], preferred_element_type=jnp.float32)
        s_max = jnp.max(scores, axis=-1, keepdims=True)
        probs = jnp.exp(scores - s_max)
        e_ids = lax.broadcasted_iota(jnp.int32, (N_TOK, E_TOTAL), 1)
        mask = jnp.logical_or(e_ids == idx_ref[:, 0:1],
                              e_ids == idx_ref[:, 1:2]).astype(jnp.float32)
        w = probs * mask
        gates = w / jnp.sum(w, axis=-1, keepdims=True)

        row = lax.broadcasted_iota(jnp.int32, (E_TOTAL, E_LOCAL), 0)
        col = lax.broadcasted_iota(jnp.int32, (E_TOTAL, E_LOCAL), 1)
        sel = (row == p * E_LOCAL + col).astype(jnp.float32)
        gates_ref[:, :] = jnp.dot(gates, sel, preferred_element_type=jnp.float32)

        def partial_chunk(c):
            start = c * CHUNK
            xs = x_ref[pl.ds(start, CHUNK), :]
            g = gates_ref[pl.ds(start, CHUNK), :]
            acc = jnp.zeros((CHUNK, D_MODEL), jnp.float32)
            for j in range(E_LOCAL):
                acc = acc + jnp.dot(xs * g[:, j:j + 1], ew_ref[j],
                                    preferred_element_type=jnp.float32)
            return acc

        for s in range(N_DEV - 1):
            c = lax.rem(p + N_DEV - 1 - s, N_DEV)
            val = partial_chunk(c)
            if s == 0:
                stage_ref[:, :] = val
            else:
                stage_ref[:, :] = val + comm_ref[s - 1]
            rdma = pltpu.make_async_remote_copy(
                src_ref=stage_ref,
                dst_ref=comm_ref.at[s],
                send_sem=send_sems.at[s],
                recv_sem=recv_sems.at[s],
                device_id=(right,),
                device_id_type=pl.DeviceIdType.MESH,
            )
            rdma.start()
            rdma.wait()

        out_ref[:, :] = comm_ref[N_DEV - 2] + partial_chunk(p)

    return pl.pallas_call(
        body,
        out_shape=jax.ShapeDtypeStruct((CHUNK, D_MODEL), jnp.float32),
        in_specs=[pl.BlockSpec(memory_space=pltpu.VMEM)] * 4,
        out_specs=pl.BlockSpec(memory_space=pltpu.VMEM),
        scratch_shapes=[
            pltpu.VMEM((N_DEV - 1, CHUNK, D_MODEL), jnp.float32),
            pltpu.VMEM((CHUNK, D_MODEL), jnp.float32),
            pltpu.VMEM((N_TOK, E_LOCAL), jnp.float32),
            pltpu.SemaphoreType.DMA((N_DEV - 1,)),
            pltpu.SemaphoreType.DMA((N_DEV - 1,)),
        ],
        compiler_params=pltpu.CompilerParams(
            collective_id=0,
            vmem_limit_bytes=60 * 1024 * 1024,
        ),
    )(x, router_W, route_idx, expert_W)


# device time: 128304 ns/iter; 1.4901x vs baseline; 1.4901x over previous
import jax
import jax.numpy as jnp
from jax import lax
from jax.experimental import pallas as pl
from jax.experimental.pallas import tpu as pltpu

N_DEV = 8
N_TOK = 2048
D_MODEL = 1024
E_LOCAL = 8
E_TOTAL = 64
CHUNK = N_TOK // N_DEV


def kernel(x, router_W, route_idx, expert_W):
    def body(x_ref, rw_ref, idx_ref, ew_ref, out_ref,
             comm_ref, stage_ref, gates_ref, send_sems, recv_sems):
        p = lax.axis_index("i")
        left = lax.rem(p + N_DEV - 1, N_DEV)
        right = lax.rem(p + 1, N_DEV)

        barrier = pltpu.get_barrier_semaphore()
        pl.semaphore_signal(barrier, inc=1, device_id=(left,),
                            device_id_type=pl.DeviceIdType.MESH)
        pl.semaphore_signal(barrier, inc=1, device_id=(right,),
                            device_id_type=pl.DeviceIdType.MESH)
        pl.semaphore_wait(barrier, 2)

        x_all = x_ref[:, :]
        scores = jnp.dot(x_all, rw_ref[:, :], preferred_element_type=jnp.float32)
        s_max = jnp.max(scores, axis=-1, keepdims=True)
        probs = jnp.exp(scores - s_max)
        e_ids = lax.broadcasted_iota(jnp.int32, (N_TOK, E_TOTAL), 1)
        mask = jnp.logical_or(e_ids == idx_ref[:, 0:1],
                              e_ids == idx_ref[:, 1:2]).astype(jnp.float32)
        w = probs * mask
        gates = w / jnp.sum(w, axis=-1, keepdims=True)

        row = lax.broadcasted_iota(jnp.int32, (E_TOTAL, E_LOCAL), 0)
        col = lax.broadcasted_iota(jnp.int32, (E_TOTAL, E_LOCAL), 1)
        sel = (row == p * E_LOCAL + col).astype(jnp.float32)
        gates_ref[:, :] = jnp.dot(gates, sel, preferred_element_type=jnp.float32)

        def partial_chunk(c):
            start = c * CHUNK
            xs = x_ref[pl.ds(start, CHUNK), :]
            g = gates_ref[pl.ds(start, CHUNK), :]
            acc = jnp.zeros((CHUNK, D_MODEL), jnp.float32)
            for j in range(E_LOCAL):
                acc = acc + jnp.dot(xs * g[:, j:j + 1], ew_ref[j],
                                    preferred_element_type=jnp.float32)
            return acc

        rdmas = []
        for s in range(N_DEV - 1):
            c = lax.rem(p + N_DEV - 1 - s, N_DEV)
            val = partial_chunk(c)
            if s >= 1:
                rdmas[s - 1].wait_recv()
            if s >= 2:
                rdmas[s - 2].wait_send()
            slot = s % 2
            if s == 0:
                stage_ref[slot] = val
            else:
                stage_ref[slot] = val + comm_ref[s - 1]
            rdma = pltpu.make_async_remote_copy(
                src_ref=stage_ref.at[slot],
                dst_ref=comm_ref.at[s],
                send_sem=send_sems.at[s],
                recv_sem=recv_sems.at[s],
                device_id=(right,),
                device_id_type=pl.DeviceIdType.MESH,
            )
            rdma.start()
            rdmas.append(rdma)

        val_p = partial_chunk(p)
        rdmas[N_DEV - 2].wait_recv()
        out_ref[:, :] = comm_ref[N_DEV - 2] + val_p
        rdmas[N_DEV - 3].wait_send()
        rdmas[N_DEV - 2].wait_send()

    return pl.pallas_call(
        body,
        out_shape=jax.ShapeDtypeStruct((CHUNK, D_MODEL), jnp.float32),
        in_specs=[pl.BlockSpec(memory_space=pltpu.VMEM)] * 4,
        out_specs=pl.BlockSpec(memory_space=pltpu.VMEM),
        scratch_shapes=[
            pltpu.VMEM((N_DEV - 1, CHUNK, D_MODEL), jnp.float32),
            pltpu.VMEM((2, CHUNK, D_MODEL), jnp.float32),
            pltpu.VMEM((N_TOK, E_LOCAL), jnp.float32),
            pltpu.SemaphoreType.DMA((N_DEV - 1,)),
            pltpu.SemaphoreType.DMA((N_DEV - 1,)),
        ],
        compiler_params=pltpu.CompilerParams(
            collective_id=0,
            vmem_limit_bytes=60 * 1024 * 1024,
        ),
    )(x, router_W, route_idx, expert_W)
